# baseline (speedup 1.0000x reference)
import functools

import jax
import jax.numpy as jnp
import numpy as np
from jax import lax
from jax.experimental import pallas as pl
from jax.experimental.pallas import tpu as pltpu
from jax.experimental.pallas import tpu_sc as plsc

_FIELD_DIMS = [100000] * 26
_OFFS = np.array((0, *np.cumsum(_FIELD_DIMS)[:-1]), dtype=np.int32)
_B = 4096
_NF = 26
_D = 16
_EPS = 1e-5

_NC = 2
_NS = 16
_NW = _NC * _NS
_RPW = _B // _NW
_EPW = _RPW * _NF * _D
_LPW = _RPW * _NF
_CHUNK = 128
_NCH = _EPW // _CHUNK


def _sc_gather_body(idx_hbm, emb_hbm, lin_hbm, emb_out, lin_out,
                    idx_v, list_v, ebuf, lbuf, sem_e, sem_l):
    wid = lax.axis_index("s") * _NC + lax.axis_index("c")
    cbase = pl.multiple_of(wid * _RPW, 8)
    pltpu.sync_copy(idx_hbm.at[:, pl.ds(cbase, _RPW)], idx_v)

    iota = lax.iota(jnp.int32, 16)
    v_rows = _NF * _FIELD_DIMS[0]

    def build_fg(k, carry):
        f = k // 8
        g = k - f * 8
        w = idx_v[f, pl.ds(g * 16, 16)]
        pos0 = iota * (_NF * _D) + (g * 16 * _NF + f) * _D
        for d in range(_D):
            plsc.store_scatter(list_v, [pos0 + d], w + d * v_rows)
        return carry

    lax.fori_loop(0, _NF * 8, build_fg, 0)

    def fire(t, carry):
        off = pl.multiple_of(t * _CHUNK, 8)
        pltpu.make_async_copy(
            emb_hbm.at[list_v.at[pl.ds(off, _CHUNK)]],
            ebuf.at[pl.ds(off, _CHUNK)],
            sem_e,
        ).start()
        return carry

    lax.fori_loop(0, _NCH, fire, 0)

    for f in range(_NF):
        pltpu.make_async_copy(
            lin_hbm.at[idx_v.at[f]],
            lbuf.at[pl.ds(f * _RPW, _RPW)],
            sem_l,
        ).start()

    pltpu.make_async_copy(emb_hbm.at[pl.ds(0, _EPW)], ebuf, sem_e).wait()
    pltpu.make_async_copy(lin_hbm.at[pl.ds(0, _LPW)], lbuf, sem_l).wait()

    ebase = pl.multiple_of(wid * _EPW, 8)
    lbase = pl.multiple_of(wid * _LPW, 8)
    pltpu.sync_copy(ebuf, emb_out.at[pl.ds(ebase, _EPW)])
    pltpu.sync_copy(lbuf, lin_out.at[pl.ds(lbase, _LPW)])


@functools.lru_cache(maxsize=1)
def _make_sc_gather():
    mesh = plsc.VectorSubcoreMesh(
        core_axis_name="c", subcore_axis_name="s",
        num_cores=_NC, num_subcores=_NS,
    )
    return pl.kernel(
        _sc_gather_body,
        out_type=(
            jax.ShapeDtypeStruct((_NW * _EPW,), jnp.float32),
            jax.ShapeDtypeStruct((_NW * _LPW,), jnp.float32),
        ),
        mesh=mesh,
        compiler_params=pltpu.CompilerParams(needs_layout_passes=False),
        scratch_types=[
            pltpu.VMEM((_NF, _RPW), jnp.int32),
            pltpu.VMEM((_EPW,), jnp.int32),
            pltpu.VMEM((_EPW,), jnp.float32),
            pltpu.VMEM((_LPW,), jnp.float32),
            pltpu.SemaphoreType.DMA,
            pltpu.SemaphoreType.DMA,
        ],
    )


_RB = 512


def _tc_head(emb_ref, linv_ref, w0_ref, b0_ref, w1_ref, b1_ref, wo_ref,
             a_ref, d_ref):
    e = emb_ref[...]
    s = e[:, 0:_D]
    q = s * s
    for f in range(1, _NF):
        c = e[:, f * _D:(f + 1) * _D]
        s = s + c
        q = q + c * c
    inter = 0.5 * jnp.sum(s * s - q, axis=1, keepdims=True)
    lin = jnp.sum(linv_ref[...], axis=1, keepdims=True)
    a_ref[...] = lin + inter
    h = jnp.dot(e, w0_ref[...], preferred_element_type=jnp.float32)
    h = jnp.maximum(h + b0_ref[...], 0.0)
    h = jnp.dot(h, w1_ref[...], preferred_element_type=jnp.float32)
    h = jnp.maximum(h + b1_ref[...], 0.0)
    d_ref[...] = jnp.dot(h, wo_ref[...], preferred_element_type=jnp.float32)


def _tc_out(a_ref, dt_ref, out_ref):
    out_ref[...] = jax.nn.sigmoid(a_ref[...] + dt_ref[...])


def kernel(x, W_emb, W_lin, lin_bias, W0, b0, g0, be0, rm0, rv0,
           W1, b1, g1, be1, rm1, rv1, Wout, bout):
    xi2 = x.T + jnp.asarray(_OFFS, dtype=x.dtype)[:, None]
    one = 1.0 + 0.0 * lin_bias[0]
    emb1 = (W_emb.T * one).reshape(-1)
    lin1 = W_lin.reshape(-1)

    inv0 = g0 / jnp.sqrt(rv0 + _EPS)
    w0t = (W0 * inv0[:, None]).T
    b0f = ((b0 - rm0) * inv0 + be0).reshape(1, -1)
    inv1 = g1 / jnp.sqrt(rv1 + _EPS)
    w1t = (W1 * inv1[:, None]).T
    b1f = ((b1 - rm1) * inv1 + be1).reshape(1, -1)
    wot = Wout.T
    bias_all = (bout + lin_bias)[0]

    emb_flat, lin_flat = _make_sc_gather()(xi2, emb1, lin1)
    emb2d = emb_flat.reshape(_B, _NF * _D)
    linv = (lin_flat.reshape(_NW, _NF, _RPW)
            .transpose(0, 2, 1).reshape(_B, _NF))

    n_blk = _B // _RB
    a, d = pl.pallas_call(
        _tc_head,
        grid=(n_blk,),
        in_specs=[
            pl.BlockSpec((_RB, _NF * _D), lambda i: (i, 0)),
            pl.BlockSpec((_RB, _NF), lambda i: (i, 0)),
            pl.BlockSpec((_NF * _D, 128), lambda i: (0, 0)),
            pl.BlockSpec((1, 128), lambda i: (0, 0)),
            pl.BlockSpec((128, 64), lambda i: (0, 0)),
            pl.BlockSpec((1, 64), lambda i: (0, 0)),
            pl.BlockSpec((64, 1), lambda i: (0, 0)),
        ],
        out_specs=[
            pl.BlockSpec((_RB, 1), lambda i: (i, 0)),
            pl.BlockSpec((_RB, 1), lambda i: (i, 0)),
        ],
        out_shape=[
            jax.ShapeDtypeStruct((_B, 1), jnp.float32),
            jax.ShapeDtypeStruct((_B, 1), jnp.float32),
        ],
    )(emb2d, linv, w0t, b0f, w1t, b1f, wot)

    dt = (d[:, 0] + bias_all).reshape(1, _B)

    out = pl.pallas_call(
        _tc_out,
        grid=(n_blk,),
        in_specs=[
            pl.BlockSpec((_RB, 1), lambda i: (i, 0)),
            pl.BlockSpec((1, _B), lambda i: (0, 0)),
        ],
        out_specs=pl.BlockSpec((_RB, _B), lambda i: (i, 0)),
        out_shape=jax.ShapeDtypeStruct((_B, _B), jnp.float32),
    )(a, dt)
    return out


# speedup vs baseline: 1.6569x; 1.6569x over previous
import functools

import jax
import jax.numpy as jnp
import numpy as np
from jax import lax
from jax.experimental import pallas as pl
from jax.experimental.pallas import tpu as pltpu
from jax.experimental.pallas import tpu_sc as plsc

_FIELD_DIMS = [100000] * 26
_OFFS = np.array((0, *np.cumsum(_FIELD_DIMS)[:-1]), dtype=np.int32)
_B = 4096
_NF = 26
_D = 16
_EPS = 1e-5

_NC = 2
_NS = 16
_NW = _NC * _NS
_RPW = _B // _NW
_EPW = _RPW * _NF * _D
_LPW = _RPW * _NF
_CHUNK = 128
_NCH = _EPW // _CHUNK


def _sc_gather_body(idx_hbm, *refs):
    planes = refs[:_D]
    lin_hbm = refs[_D]
    emb_out, lin_out = refs[_D + 1], refs[_D + 2]
    idx_v, ebuf, lbuf, sem_e, sem_l = refs[_D + 3:]

    wid = lax.axis_index("s") * _NC + lax.axis_index("c")
    cbase = pl.multiple_of(wid * _RPW, 8)
    pltpu.sync_copy(idx_hbm.at[:, pl.ds(cbase, _RPW)], idx_v)

    def fire(f, carry):
        idx_row = idx_v.at[f]
        for d in range(_D):
            pltpu.make_async_copy(
                planes[d].at[idx_row], ebuf.at[f * _D + d], sem_e
            ).start()
        pltpu.make_async_copy(
            lin_hbm.at[idx_row], lbuf.at[f], sem_l
        ).start()
        return carry

    lax.fori_loop(0, _NF, fire, 0)

    pltpu.make_async_copy(
        emb_out.at[:, pl.ds(0, _RPW)], ebuf, sem_e
    ).wait()
    pltpu.make_async_copy(
        lin_out.at[:, pl.ds(0, _RPW)], lbuf, sem_l
    ).wait()

    pltpu.sync_copy(ebuf, emb_out.at[:, pl.ds(cbase, _RPW)])
    pltpu.sync_copy(lbuf, lin_out.at[:, pl.ds(cbase, _RPW)])


@functools.lru_cache(maxsize=1)
def _make_sc_gather():
    mesh = plsc.VectorSubcoreMesh(
        core_axis_name="c", subcore_axis_name="s",
        num_cores=_NC, num_subcores=_NS,
    )
    return pl.kernel(
        _sc_gather_body,
        out_type=(
            jax.ShapeDtypeStruct((_NF * _D, _B), jnp.float32),
            jax.ShapeDtypeStruct((_NF, _B), jnp.float32),
        ),
        mesh=mesh,
        compiler_params=pltpu.CompilerParams(needs_layout_passes=False),
        scratch_types=[
            pltpu.VMEM((_NF, _RPW), jnp.int32),
            pltpu.VMEM((_NF * _D, _RPW), jnp.float32),
            pltpu.VMEM((_NF, _RPW), jnp.float32),
            pltpu.SemaphoreType.DMA,
            pltpu.SemaphoreType.DMA,
        ],
    )


def _tc_split(emb_ref, *out_refs):
    e = emb_ref[...]
    for d in range(_D):
        out_refs[d][...] = e[d:d + 1, :]


_CB = 16384


@functools.lru_cache(maxsize=1)
def _make_plane_split():
    v = _NF * _FIELD_DIMS[0]
    n_blk = (v + _CB - 1) // _CB
    return pl.pallas_call(
        _tc_split,
        grid=(n_blk,),
        in_specs=[pl.BlockSpec((_D, _CB), lambda i: (0, i))],
        out_specs=[pl.BlockSpec((1, _CB), lambda i: (0, i))] * _D,
        out_shape=[jax.ShapeDtypeStruct((1, v), jnp.float32)] * _D,
    )


_RB = 512


def _tc_head(embt_ref, linv_ref, w0_ref, b0_ref, w1_ref, b1_ref, wo_ref,
             a_ref, d_ref):
    e = embt_ref[...]
    s = e[0:_D, :]
    q = s * s
    for f in range(1, _NF):
        c = e[f * _D:(f + 1) * _D, :]
        s = s + c
        q = q + c * c
    inter = 0.5 * jnp.sum(s * s - q, axis=0, keepdims=True)
    lin = jnp.sum(linv_ref[...], axis=0, keepdims=True)
    a_ref[...] = lin + inter
    h = jnp.dot(w0_ref[...], e, preferred_element_type=jnp.float32)
    h = jnp.maximum(h + b0_ref[...], 0.0)
    h = jnp.dot(w1_ref[...], h, preferred_element_type=jnp.float32)
    h = jnp.maximum(h + b1_ref[...], 0.0)
    d_ref[...] = jnp.dot(wo_ref[...], h, preferred_element_type=jnp.float32)


def _tc_out(a_ref, dt_ref, out_ref):
    out_ref[...] = 1.0 / (1.0 + a_ref[...] * dt_ref[...])


def kernel(x, W_emb, W_lin, lin_bias, W0, b0, g0, be0, rm0, rv0,
           W1, b1, g1, be1, rm1, rv1, Wout, bout):
    xi2 = x.T + jnp.asarray(_OFFS, dtype=x.dtype)[:, None]
    lin1 = W_lin.reshape(-1)

    inv0 = g0 / jnp.sqrt(rv0 + _EPS)
    w0f = W0 * inv0[:, None]
    b0f = ((b0 - rm0) * inv0 + be0).reshape(-1, 1)
    inv1 = g1 / jnp.sqrt(rv1 + _EPS)
    w1f = W1 * inv1[:, None]
    b1f = ((b1 - rm1) * inv1 + be1).reshape(-1, 1)
    bias_all = (bout + lin_bias)[0]

    planes = _make_plane_split()(W_emb.T)
    planes = [p.reshape(-1) for p in planes]

    embt, linv = _make_sc_gather()(xi2, *planes, lin1)

    n_blk = _B // _RB
    a, d = pl.pallas_call(
        _tc_head,
        grid=(n_blk,),
        in_specs=[
            pl.BlockSpec((_NF * _D, _RB), lambda i: (0, i)),
            pl.BlockSpec((_NF, _RB), lambda i: (0, i)),
            pl.BlockSpec((128, _NF * _D), lambda i: (0, 0)),
            pl.BlockSpec((128, 1), lambda i: (0, 0)),
            pl.BlockSpec((64, 128), lambda i: (0, 0)),
            pl.BlockSpec((64, 1), lambda i: (0, 0)),
            pl.BlockSpec((1, 64), lambda i: (0, 0)),
        ],
        out_specs=[
            pl.BlockSpec((1, _RB), lambda i: (0, i)),
            pl.BlockSpec((1, _RB), lambda i: (0, i)),
        ],
        out_shape=[
            jax.ShapeDtypeStruct((1, _B), jnp.float32),
            jax.ShapeDtypeStruct((1, _B), jnp.float32),
        ],
    )(embt, linv, w0f, b0f, w1f, b1f, Wout)

    ue = jnp.exp(-a).reshape(_B, 1)
    dt = jnp.exp(-(d + bias_all))

    out = pl.pallas_call(
        _tc_out,
        grid=(n_blk,),
        in_specs=[
            pl.BlockSpec((_RB, 1), lambda i: (i, 0)),
            pl.BlockSpec((1, _B), lambda i: (0, 0)),
        ],
        out_specs=pl.BlockSpec((_RB, _B), lambda i: (i, 0)),
        out_shape=jax.ShapeDtypeStruct((_B, _B), jnp.float32),
    )(ue, dt)
    return out


# speedup vs baseline: 8.9355x; 5.3929x over previous
import functools

import jax
import jax.numpy as jnp
import numpy as np
from jax import lax
from jax.experimental import pallas as pl
from jax.experimental.pallas import tpu as pltpu
from jax.experimental.pallas import tpu_sc as plsc

_FIELD_DIMS = [100000] * 26
_OFFS = np.array((0, *np.cumsum(_FIELD_DIMS)[:-1]), dtype=np.int32)
_B = 4096
_NF = 26
_D = 16
_EPS = 1e-5

_NC = 2
_NS = 16
_NW = _NC * _NS
_RPW = _B // _NW
_EPW = _RPW * _NF * _D
_LPW = _RPW * _NF
_CHUNK = 128
_NCH = _EPW // _CHUNK


def _sc_gather_body(idx_hbm, *refs):
    planes = refs[:_D]
    lin_hbm = refs[_D]
    emb_out, lin_out = refs[_D + 1], refs[_D + 2]
    idx_v, ebuf, lbuf, sem_e, sem_l = refs[_D + 3:]

    wid = lax.axis_index("s") * _NC + lax.axis_index("c")
    cbase = pl.multiple_of(wid * _RPW, 8)
    pltpu.sync_copy(idx_hbm.at[:, pl.ds(cbase, _RPW)], idx_v)

    def fire(f, carry):
        idx_row = idx_v.at[f]
        for d in range(_D):
            pltpu.make_async_copy(
                planes[d].at[idx_row], ebuf.at[f * _D + d], sem_e
            ).start()
        pltpu.make_async_copy(
            lin_hbm.at[idx_row], lbuf.at[f], sem_l
        ).start()
        return carry

    lax.fori_loop(0, _NF, fire, 0)

    pltpu.make_async_copy(
        emb_out.at[:, pl.ds(0, _RPW)], ebuf, sem_e
    ).wait()
    pltpu.make_async_copy(
        lin_out.at[:, pl.ds(0, _RPW)], lbuf, sem_l
    ).wait()

    pltpu.sync_copy(ebuf, emb_out.at[:, pl.ds(cbase, _RPW)])
    pltpu.sync_copy(lbuf, lin_out.at[:, pl.ds(cbase, _RPW)])


@functools.lru_cache(maxsize=1)
def _make_sc_gather():
    mesh = plsc.VectorSubcoreMesh(
        core_axis_name="c", subcore_axis_name="s",
        num_cores=_NC, num_subcores=_NS,
    )
    return pl.kernel(
        _sc_gather_body,
        out_type=(
            jax.ShapeDtypeStruct((_NF * _D, _B), jnp.float32),
            jax.ShapeDtypeStruct((_NF, _B), jnp.float32),
        ),
        mesh=mesh,
        compiler_params=pltpu.CompilerParams(needs_layout_passes=False),
        scratch_types=[
            pltpu.VMEM((_NF, _RPW), jnp.int32),
            pltpu.VMEM((_NF * _D, _RPW), jnp.float32),
            pltpu.VMEM((_NF, _RPW), jnp.float32),
            pltpu.SemaphoreType.DMA,
            pltpu.SemaphoreType.DMA,
        ],
    )


def _tc_split(emb_ref, *out_refs):
    e = emb_ref[...]
    for d in range(_D):
        out_refs[d][...] = e[d, :]


_CB = 16384


@functools.lru_cache(maxsize=1)
def _make_plane_split():
    v = _NF * _FIELD_DIMS[0]
    n_blk = (v + _CB - 1) // _CB
    return pl.pallas_call(
        _tc_split,
        grid=(n_blk,),
        in_specs=[pl.BlockSpec((_D, _CB), lambda i: (0, i))],
        out_specs=[pl.BlockSpec((_CB,), lambda i: (i,))] * _D,
        out_shape=[jax.ShapeDtypeStruct((v,), jnp.float32)] * _D,
    )


_RB = 512


def _tc_head(embt_ref, linv_ref, w0_ref, b0_ref, w1_ref, b1_ref, wo_ref,
             a_ref, d_ref):
    e = embt_ref[...]
    s = e[0:_D, :]
    q = s * s
    for f in range(1, _NF):
        c = e[f * _D:(f + 1) * _D, :]
        s = s + c
        q = q + c * c
    inter = 0.5 * jnp.sum(s * s - q, axis=0, keepdims=True)
    lin = jnp.sum(linv_ref[...], axis=0, keepdims=True)
    a_ref[...] = lin + inter
    h = jnp.dot(w0_ref[...], e, preferred_element_type=jnp.float32)
    h = jnp.maximum(h + b0_ref[...], 0.0)
    h = jnp.dot(w1_ref[...], h, preferred_element_type=jnp.float32)
    h = jnp.maximum(h + b1_ref[...], 0.0)
    d_ref[...] = jnp.dot(wo_ref[...], h, preferred_element_type=jnp.float32)


def _tc_out(a_ref, dt_ref, out_ref):
    out_ref[...] = 1.0 / (1.0 + a_ref[...] * dt_ref[...])


def kernel(x, W_emb, W_lin, lin_bias, W0, b0, g0, be0, rm0, rv0,
           W1, b1, g1, be1, rm1, rv1, Wout, bout):
    xi2 = x.T + jnp.asarray(_OFFS, dtype=x.dtype)[:, None]
    lin1 = W_lin.reshape(-1)

    inv0 = g0 / jnp.sqrt(rv0 + _EPS)
    w0f = W0 * inv0[:, None]
    b0f = ((b0 - rm0) * inv0 + be0).reshape(-1, 1)
    inv1 = g1 / jnp.sqrt(rv1 + _EPS)
    w1f = W1 * inv1[:, None]
    b1f = ((b1 - rm1) * inv1 + be1).reshape(-1, 1)
    bias_all = (bout + lin_bias)[0]

    planes = _make_plane_split()(W_emb.T)

    embt, linv = _make_sc_gather()(xi2, *planes, lin1)

    n_blk = _B // _RB
    a, d = pl.pallas_call(
        _tc_head,
        grid=(n_blk,),
        in_specs=[
            pl.BlockSpec((_NF * _D, _RB), lambda i: (0, i)),
            pl.BlockSpec((_NF, _RB), lambda i: (0, i)),
            pl.BlockSpec((128, _NF * _D), lambda i: (0, 0)),
            pl.BlockSpec((128, 1), lambda i: (0, 0)),
            pl.BlockSpec((64, 128), lambda i: (0, 0)),
            pl.BlockSpec((64, 1), lambda i: (0, 0)),
            pl.BlockSpec((1, 64), lambda i: (0, 0)),
        ],
        out_specs=[
            pl.BlockSpec((1, _RB), lambda i: (0, i)),
            pl.BlockSpec((1, _RB), lambda i: (0, i)),
        ],
        out_shape=[
            jax.ShapeDtypeStruct((1, _B), jnp.float32),
            jax.ShapeDtypeStruct((1, _B), jnp.float32),
        ],
    )(embt, linv, w0f, b0f, w1f, b1f, Wout)

    ue = jnp.exp(-a).reshape(_B, 1)
    dt = jnp.exp(-(d + bias_all))

    out = pl.pallas_call(
        _tc_out,
        grid=(n_blk,),
        in_specs=[
            pl.BlockSpec((_RB, 1), lambda i: (i, 0)),
            pl.BlockSpec((1, _B), lambda i: (0, 0)),
        ],
        out_specs=pl.BlockSpec((_RB, _B), lambda i: (i, 0)),
        out_shape=jax.ShapeDtypeStruct((_B, _B), jnp.float32),
    )(ue, dt)
    return out


# speedup vs baseline: 12.1588x; 1.3607x over previous
import functools

import jax
import jax.numpy as jnp
import numpy as np
from jax import lax
from jax.experimental import pallas as pl
from jax.experimental.pallas import tpu as pltpu
from jax.experimental.pallas import tpu_sc as plsc

_FIELD_DIMS = [100000] * 26
_OFFS = np.array((0, *np.cumsum(_FIELD_DIMS)[:-1]), dtype=np.int32)
_B = 4096
_NF = 26
_D = 16
_EPS = 1e-5

_NC = 2
_NS = 16
_NW = _NC * _NS
_RPW = _B // _NW
_EPW = _RPW * _NF * _D
_LPW = _RPW * _NF
_CHUNK = 128
_NCH = _EPW // _CHUNK


def _sc_gather_body(idx_hbm, *refs):
    planes = refs[:_D]
    lin_hbm = refs[_D]
    emb_out, lin_out = refs[_D + 1], refs[_D + 2]
    idx_v, ebuf, lbuf, sem_e, sem_l = refs[_D + 3:]

    wid = lax.axis_index("s") * _NC + lax.axis_index("c")
    cbase = pl.multiple_of(wid * _RPW, 8)
    pltpu.sync_copy(idx_hbm.at[:, pl.ds(cbase, _RPW)], idx_v)

    def fire(f, carry):
        idx_row = idx_v.at[f]
        for d in range(_D):
            pltpu.make_async_copy(
                planes[d].at[idx_row], ebuf.at[f * _D + d], sem_e
            ).start()
        pltpu.make_async_copy(
            lin_hbm.at[idx_row], lbuf.at[f], sem_l
        ).start()
        return carry

    lax.fori_loop(0, _NF, fire, 0)

    pltpu.make_async_copy(
        emb_out.at[:, pl.ds(0, _RPW)], ebuf, sem_e
    ).wait()
    pltpu.make_async_copy(
        lin_out.at[:, pl.ds(0, _RPW)], lbuf, sem_l
    ).wait()

    pltpu.sync_copy(ebuf, emb_out.at[:, pl.ds(cbase, _RPW)])
    pltpu.sync_copy(lbuf, lin_out.at[:, pl.ds(cbase, _RPW)])


@functools.lru_cache(maxsize=1)
def _make_sc_gather():
    mesh = plsc.VectorSubcoreMesh(
        core_axis_name="c", subcore_axis_name="s",
        num_cores=_NC, num_subcores=_NS,
    )
    return pl.kernel(
        _sc_gather_body,
        out_type=(
            jax.ShapeDtypeStruct((_NF * _D, _B), jnp.float32),
            jax.ShapeDtypeStruct((_NF, _B), jnp.float32),
        ),
        mesh=mesh,
        compiler_params=pltpu.CompilerParams(needs_layout_passes=False),
        scratch_types=[
            pltpu.VMEM((_NF, _RPW), jnp.int32),
            pltpu.VMEM((_NF * _D, _RPW), jnp.float32),
            pltpu.VMEM((_NF, _RPW), jnp.float32),
            pltpu.SemaphoreType.DMA,
            pltpu.SemaphoreType.DMA,
        ],
    )


def _tc_split(emb_ref, lin_ref, *out_refs):
    e = emb_ref[...]
    for d in range(_D):
        out_refs[d][...] = e[d, :]
    out_refs[_D][...] = lin_ref[0, :]


_CB = 16384


@functools.lru_cache(maxsize=1)
def _make_plane_split():
    v = _NF * _FIELD_DIMS[0]
    n_blk = (v + _CB - 1) // _CB
    return pl.pallas_call(
        _tc_split,
        grid=(n_blk,),
        in_specs=[
            pl.BlockSpec((_D, _CB), lambda i: (0, i)),
            pl.BlockSpec((1, _CB), lambda i: (0, i)),
        ],
        out_specs=[pl.BlockSpec((_CB,), lambda i: (i,))] * (_D + 1),
        out_shape=[jax.ShapeDtypeStruct((v,), jnp.float32)] * (_D + 1),
    )


_RB = 512


def _tc_head(embt_ref, linv_ref, w0_ref, b0_ref, w1_ref, b1_ref, wo_ref,
             a_ref, d_ref):
    e = embt_ref[...]
    s = e[0:_D, :]
    q = s * s
    for f in range(1, _NF):
        c = e[f * _D:(f + 1) * _D, :]
        s = s + c
        q = q + c * c
    inter = 0.5 * jnp.sum(s * s - q, axis=0, keepdims=True)
    lin = jnp.sum(linv_ref[...], axis=0, keepdims=True)
    a_ref[...] = lin + inter
    h = jnp.dot(w0_ref[...], e, preferred_element_type=jnp.float32)
    h = jnp.maximum(h + b0_ref[...], 0.0)
    h = jnp.dot(w1_ref[...], h, preferred_element_type=jnp.float32)
    h = jnp.maximum(h + b1_ref[...], 0.0)
    d_ref[...] = jnp.dot(wo_ref[...], h, preferred_element_type=jnp.float32)


def _tc_out(a_ref, dt_ref, out_ref):
    out_ref[...] = 1.0 / (1.0 + a_ref[...] * dt_ref[...])


def kernel(x, W_emb, W_lin, lin_bias, W0, b0, g0, be0, rm0, rv0,
           W1, b1, g1, be1, rm1, rv1, Wout, bout):
    xi2 = x.T + jnp.asarray(_OFFS, dtype=x.dtype)[:, None]

    inv0 = g0 / jnp.sqrt(rv0 + _EPS)
    w0f = W0 * inv0[:, None]
    b0f = ((b0 - rm0) * inv0 + be0).reshape(-1, 1)
    inv1 = g1 / jnp.sqrt(rv1 + _EPS)
    w1f = W1 * inv1[:, None]
    b1f = ((b1 - rm1) * inv1 + be1).reshape(-1, 1)
    bias_all = (bout + lin_bias)[0]

    splits = _make_plane_split()(W_emb.T, W_lin.T)

    embt, linv = _make_sc_gather()(xi2, *splits)

    n_blk = _B // _RB
    a, d = pl.pallas_call(
        _tc_head,
        grid=(n_blk,),
        in_specs=[
            pl.BlockSpec((_NF * _D, _RB), lambda i: (0, i)),
            pl.BlockSpec((_NF, _RB), lambda i: (0, i)),
            pl.BlockSpec((128, _NF * _D), lambda i: (0, 0)),
            pl.BlockSpec((128, 1), lambda i: (0, 0)),
            pl.BlockSpec((64, 128), lambda i: (0, 0)),
            pl.BlockSpec((64, 1), lambda i: (0, 0)),
            pl.BlockSpec((1, 64), lambda i: (0, 0)),
        ],
        out_specs=[
            pl.BlockSpec((1, _RB), lambda i: (0, i)),
            pl.BlockSpec((1, _RB), lambda i: (0, i)),
        ],
        out_shape=[
            jax.ShapeDtypeStruct((1, _B), jnp.float32),
            jax.ShapeDtypeStruct((1, _B), jnp.float32),
        ],
    )(embt, linv, w0f, b0f, w1f, b1f, Wout)

    ue = jnp.exp(-a).reshape(_B, 1)
    dt = jnp.exp(-(d + bias_all))

    out = pl.pallas_call(
        _tc_out,
        grid=(n_blk,),
        in_specs=[
            pl.BlockSpec((_RB, 1), lambda i: (i, 0)),
            pl.BlockSpec((1, _B), lambda i: (0, 0)),
        ],
        out_specs=pl.BlockSpec((_RB, _B), lambda i: (i, 0)),
        out_shape=jax.ShapeDtypeStruct((_B, _B), jnp.float32),
    )(ue, dt)
    return out


# speedup vs baseline: 13.5194x; 1.1119x over previous
import functools

import jax
import jax.numpy as jnp
import numpy as np
from jax import lax
from jax.experimental import pallas as pl
from jax.experimental.pallas import tpu as pltpu
from jax.experimental.pallas import tpu_sc as plsc

_FIELD_DIMS = [100000] * 26
_OFFS = np.array((0, *np.cumsum(_FIELD_DIMS)[:-1]), dtype=np.int32)
_B = 4096
_NF = 26
_D = 16
_EPS = 1e-5

_NC = 2
_NS = 16
_NW = _NC * _NS
_RPW = _B // _NW
_EPW = _RPW * _NF * _D
_LPW = _RPW * _NF
_CHUNK = 128
_NCH = _EPW // _CHUNK


def _sc_gather_body(nf, idx_hbm, *refs):
    planes = refs[:_D]
    lin_hbm = refs[_D]
    emb_out, lin_out = refs[_D + 1], refs[_D + 2]
    idx_v, ebuf, lbuf, sem_e, sem_l = refs[_D + 3:]

    wid = lax.axis_index("s") * _NC + lax.axis_index("c")
    cbase = pl.multiple_of(wid * _RPW, 8)
    pltpu.sync_copy(idx_hbm.at[:, pl.ds(cbase, _RPW)], idx_v)

    def fire(f, carry):
        idx_row = idx_v.at[f]
        for d in range(_D):
            pltpu.make_async_copy(
                planes[d].at[idx_row], ebuf.at[f * _D + d], sem_e
            ).start()
        pltpu.make_async_copy(
            lin_hbm.at[idx_row], lbuf.at[f], sem_l
        ).start()
        return carry

    lax.fori_loop(0, nf, fire, 0)

    pltpu.make_async_copy(
        emb_out.at[:, pl.ds(0, _RPW)], ebuf, sem_e
    ).wait()
    pltpu.make_async_copy(
        lin_out.at[:, pl.ds(0, _RPW)], lbuf, sem_l
    ).wait()

    pltpu.sync_copy(ebuf, emb_out.at[:, pl.ds(cbase, _RPW)])
    pltpu.sync_copy(lbuf, lin_out.at[:, pl.ds(cbase, _RPW)])


@functools.lru_cache(maxsize=4)
def _make_sc_gather(nf):
    mesh = plsc.VectorSubcoreMesh(
        core_axis_name="c", subcore_axis_name="s",
        num_cores=_NC, num_subcores=_NS,
    )
    return pl.kernel(
        functools.partial(_sc_gather_body, nf),
        out_type=(
            jax.ShapeDtypeStruct((nf * _D, _B), jnp.float32),
            jax.ShapeDtypeStruct((nf, _B), jnp.float32),
        ),
        mesh=mesh,
        compiler_params=pltpu.CompilerParams(needs_layout_passes=False),
        scratch_types=[
            pltpu.VMEM((nf, _RPW), jnp.int32),
            pltpu.VMEM((nf * _D, _RPW), jnp.float32),
            pltpu.VMEM((nf, _RPW), jnp.float32),
            pltpu.SemaphoreType.DMA,
            pltpu.SemaphoreType.DMA,
        ],
    )


def _tc_split(emb_ref, lin_ref, *out_refs):
    e = emb_ref[...]
    for d in range(_D):
        out_refs[d][...] = e[d, :]
    out_refs[_D][...] = lin_ref[0, :]


_CB = 16384
_NF_H = 13
_COL_SPLIT = _NF_H * _FIELD_DIMS[0]
_BLK0_H2 = _COL_SPLIT // _CB
_OFF_H2 = _BLK0_H2 * _CB


@functools.lru_cache(maxsize=4)
def _make_plane_split(blk0, n_blk):
    vh = n_blk * _CB
    return pl.pallas_call(
        _tc_split,
        grid=(n_blk,),
        in_specs=[
            pl.BlockSpec((_D, _CB), lambda i: (0, i + blk0)),
            pl.BlockSpec((1, _CB), lambda i: (0, i + blk0)),
        ],
        out_specs=[pl.BlockSpec((_CB,), lambda i: (i,))] * (_D + 1),
        out_shape=[jax.ShapeDtypeStruct((vh,), jnp.float32)] * (_D + 1),
    )


_RB = 512


def _tc_head(e1_ref, e2_ref, l1_ref, l2_ref, w0_ref, b0_ref, w1_ref, b1_ref,
             wo_ref, a_ref, d_ref):
    e1 = e1_ref[...]
    e2 = e2_ref[...]
    s = e1[0:_D, :]
    q = s * s
    for f in range(1, _NF_H):
        c = e1[f * _D:(f + 1) * _D, :]
        s = s + c
        q = q + c * c
    for f in range(_NF - _NF_H):
        c = e2[f * _D:(f + 1) * _D, :]
        s = s + c
        q = q + c * c
    inter = 0.5 * jnp.sum(s * s - q, axis=0, keepdims=True)
    lin = (jnp.sum(l1_ref[...], axis=0, keepdims=True)
           + jnp.sum(l2_ref[...], axis=0, keepdims=True))
    a_ref[...] = lin + inter
    w0 = w0_ref[...]
    nh = _NF_H * _D
    h = (jnp.dot(w0[:, :nh], e1, preferred_element_type=jnp.float32)
         + jnp.dot(w0[:, nh:], e2, preferred_element_type=jnp.float32))
    h = jnp.maximum(h + b0_ref[...], 0.0)
    h = jnp.dot(w1_ref[...], h, preferred_element_type=jnp.float32)
    h = jnp.maximum(h + b1_ref[...], 0.0)
    d_ref[...] = jnp.dot(wo_ref[...], h, preferred_element_type=jnp.float32)


def _tc_out(a_ref, dt_ref, out_ref):
    out_ref[...] = 1.0 / (1.0 + a_ref[...] * dt_ref[...])


def kernel(x, W_emb, W_lin, lin_bias, W0, b0, g0, be0, rm0, rv0,
           W1, b1, g1, be1, rm1, rv1, Wout, bout):
    offs = jnp.asarray(_OFFS, dtype=x.dtype)
    xta = x.T[:_NF_H] + offs[:_NF_H, None]
    xtb = x.T[_NF_H:] + (offs[_NF_H:, None] - _OFF_H2)

    inv0 = g0 / jnp.sqrt(rv0 + _EPS)
    w0f = W0 * inv0[:, None]
    b0f = ((b0 - rm0) * inv0 + be0).reshape(-1, 1)
    inv1 = g1 / jnp.sqrt(rv1 + _EPS)
    w1f = W1 * inv1[:, None]
    b1f = ((b1 - rm1) * inv1 + be1).reshape(-1, 1)
    bias_all = (bout + lin_bias)[0]

    v = _NF * _FIELD_DIMS[0]
    n_blk1 = (_COL_SPLIT + _CB - 1) // _CB
    n_blk2 = (v - _OFF_H2 + _CB - 1) // _CB
    splits1 = _make_plane_split(0, n_blk1)(W_emb.T, W_lin.T)
    splits2 = _make_plane_split(_BLK0_H2, n_blk2)(W_emb.T, W_lin.T)

    embt1, linv1 = _make_sc_gather(_NF_H)(xta, *splits1)
    embt2, linv2 = _make_sc_gather(_NF - _NF_H)(xtb, *splits2)

    n_blk = _B // _RB
    nh = _NF_H * _D
    nh2 = (_NF - _NF_H) * _D
    a, d = pl.pallas_call(
        _tc_head,
        grid=(n_blk,),
        in_specs=[
            pl.BlockSpec((nh, _RB), lambda i: (0, i)),
            pl.BlockSpec((nh2, _RB), lambda i: (0, i)),
            pl.BlockSpec((_NF_H, _RB), lambda i: (0, i)),
            pl.BlockSpec((_NF - _NF_H, _RB), lambda i: (0, i)),
            pl.BlockSpec((128, _NF * _D), lambda i: (0, 0)),
            pl.BlockSpec((128, 1), lambda i: (0, 0)),
            pl.BlockSpec((64, 128), lambda i: (0, 0)),
            pl.BlockSpec((64, 1), lambda i: (0, 0)),
            pl.BlockSpec((1, 64), lambda i: (0, 0)),
        ],
        out_specs=[
            pl.BlockSpec((1, _RB), lambda i: (0, i)),
            pl.BlockSpec((1, _RB), lambda i: (0, i)),
        ],
        out_shape=[
            jax.ShapeDtypeStruct((1, _B), jnp.float32),
            jax.ShapeDtypeStruct((1, _B), jnp.float32),
        ],
    )(embt1, embt2, linv1, linv2, w0f, b0f, w1f, b1f, Wout)

    ue = jnp.exp(-a).reshape(_B, 1)
    dt = jnp.exp(-(d + bias_all))

    out = pl.pallas_call(
        _tc_out,
        grid=(n_blk,),
        in_specs=[
            pl.BlockSpec((_RB, 1), lambda i: (i, 0)),
            pl.BlockSpec((1, _B), lambda i: (0, 0)),
        ],
        out_specs=pl.BlockSpec((_RB, _B), lambda i: (i, 0)),
        out_shape=jax.ShapeDtypeStruct((_B, _B), jnp.float32),
    )(ue, dt)
    return out


# speedup vs baseline: 13.7568x; 1.0176x over previous
import functools

import jax
import jax.numpy as jnp
import numpy as np
from jax import lax
from jax.experimental import pallas as pl
from jax.experimental.pallas import tpu as pltpu
from jax.experimental.pallas import tpu_sc as plsc

_FIELD_DIMS = [100000] * 26
_OFFS = np.array((0, *np.cumsum(_FIELD_DIMS)[:-1]), dtype=np.int32)
_B = 4096
_NF = 26
_D = 16
_EPS = 1e-5

_NC = 2
_NS = 16
_NW = _NC * _NS
_RPW = _B // _NW
_EPW = _RPW * _NF * _D
_LPW = _RPW * _NF
_CHUNK = 128
_NCH = _EPW // _CHUNK


def _sc_gather_body(nf, idx_hbm, *refs):
    planes = refs[:_D]
    lin_hbm = refs[_D]
    emb_out, lin_out = refs[_D + 1], refs[_D + 2]
    idx_v, ebuf, lbuf, sem_e, sem_l = refs[_D + 3:]

    wid = lax.axis_index("s") * _NC + lax.axis_index("c")
    cbase = pl.multiple_of(wid * _RPW, 8)
    pltpu.sync_copy(idx_hbm.at[:, pl.ds(cbase, _RPW)], idx_v)

    def fire(f, carry):
        idx_row = idx_v.at[f]
        for d in range(_D):
            pltpu.make_async_copy(
                planes[d].at[idx_row], ebuf.at[f * _D + d], sem_e
            ).start()
        pltpu.make_async_copy(
            lin_hbm.at[idx_row], lbuf.at[f], sem_l
        ).start()
        return carry

    lax.fori_loop(0, nf, fire, 0)

    pltpu.make_async_copy(
        emb_out.at[:, pl.ds(0, _RPW)], ebuf, sem_e
    ).wait()
    pltpu.make_async_copy(
        lin_out.at[:, pl.ds(0, _RPW)], lbuf, sem_l
    ).wait()

    pltpu.sync_copy(ebuf, emb_out.at[:, pl.ds(cbase, _RPW)])
    pltpu.sync_copy(lbuf, lin_out.at[:, pl.ds(cbase, _RPW)])


@functools.lru_cache(maxsize=4)
def _make_sc_gather(nf):
    mesh = plsc.VectorSubcoreMesh(
        core_axis_name="c", subcore_axis_name="s",
        num_cores=_NC, num_subcores=_NS,
    )
    return pl.kernel(
        functools.partial(_sc_gather_body, nf),
        out_type=(
            jax.ShapeDtypeStruct((nf * _D, _B), jnp.float32),
            jax.ShapeDtypeStruct((nf, _B), jnp.float32),
        ),
        mesh=mesh,
        compiler_params=pltpu.CompilerParams(needs_layout_passes=False),
        scratch_types=[
            pltpu.VMEM((nf, _RPW), jnp.int32),
            pltpu.VMEM((nf * _D, _RPW), jnp.float32),
            pltpu.VMEM((nf, _RPW), jnp.float32),
            pltpu.SemaphoreType.DMA,
            pltpu.SemaphoreType.DMA,
        ],
    )


def _tc_split(emb_ref, lin_ref, *out_refs):
    e = emb_ref[...]
    for d in range(_D):
        out_refs[d][...] = e[d, :]
    out_refs[_D][...] = lin_ref[0, :]


_CB = 16384
_QS = [(0, 7), (7, 6), (13, 7), (20, 6)]
_Q_START = [f0 * _FIELD_DIMS[0] for f0, _ in _QS]
_Q_END = [(f0 + nf) * _FIELD_DIMS[0] for f0, nf in _QS]
_Q_BLK0 = [s // _CB for s in _Q_START]
_Q_OFF = [b * _CB for b in _Q_BLK0]
_Q_NBLK = [(e - o + _CB - 1) // _CB for e, o in zip(_Q_END, _Q_OFF)]


@functools.lru_cache(maxsize=4)
def _make_plane_split(blk0, n_blk):
    vh = n_blk * _CB
    return pl.pallas_call(
        _tc_split,
        grid=(n_blk,),
        in_specs=[
            pl.BlockSpec((_D, _CB), lambda i: (0, i + blk0)),
            pl.BlockSpec((1, _CB), lambda i: (0, i + blk0)),
        ],
        out_specs=[pl.BlockSpec((_CB,), lambda i: (i,))] * (_D + 1),
        out_shape=[jax.ShapeDtypeStruct((vh,), jnp.float32)] * (_D + 1),
    )


_RB = 512


def _tc_head(e1_ref, e2_ref, e3_ref, e4_ref, l1_ref, l2_ref, l3_ref, l4_ref,
             w0_ref, b0_ref, w1_ref, b1_ref, wo_ref, a_ref, d_ref):
    e_refs = (e1_ref, e2_ref, e3_ref, e4_ref)
    l_refs = (l1_ref, l2_ref, l3_ref, l4_ref)
    w0 = w0_ref[...]
    s = None
    q = None
    lin = None
    h = None
    col = 0
    for p, (_, nf) in enumerate(_QS):
        e = e_refs[p][...]
        for f in range(nf):
            c = e[f * _D:(f + 1) * _D, :]
            s = c if s is None else s + c
            q = c * c if q is None else q + c * c
        lp = jnp.sum(l_refs[p][...], axis=0, keepdims=True)
        lin = lp if lin is None else lin + lp
        hp = jnp.dot(w0[:, col:col + nf * _D], e,
                     preferred_element_type=jnp.float32)
        h = hp if h is None else h + hp
        col += nf * _D
    inter = 0.5 * jnp.sum(s * s - q, axis=0, keepdims=True)
    a_ref[...] = lin + inter
    h = jnp.maximum(h + b0_ref[...], 0.0)
    h = jnp.dot(w1_ref[...], h, preferred_element_type=jnp.float32)
    h = jnp.maximum(h + b1_ref[...], 0.0)
    d_ref[...] = jnp.dot(wo_ref[...], h, preferred_element_type=jnp.float32)


def _tc_out(a_ref, dt_ref, out_ref):
    out_ref[...] = 1.0 / (1.0 + a_ref[...] * dt_ref[...])


def kernel(x, W_emb, W_lin, lin_bias, W0, b0, g0, be0, rm0, rv0,
           W1, b1, g1, be1, rm1, rv1, Wout, bout):
    offs = jnp.asarray(_OFFS, dtype=x.dtype)
    xts = [
        x.T[f0:f0 + nf] + (offs[f0:f0 + nf, None] - _Q_OFF[p])
        for p, (f0, nf) in enumerate(_QS)
    ]

    inv0 = g0 / jnp.sqrt(rv0 + _EPS)
    w0f = W0 * inv0[:, None]
    b0f = ((b0 - rm0) * inv0 + be0).reshape(-1, 1)
    inv1 = g1 / jnp.sqrt(rv1 + _EPS)
    w1f = W1 * inv1[:, None]
    b1f = ((b1 - rm1) * inv1 + be1).reshape(-1, 1)
    bias_all = (bout + lin_bias)[0]

    embts = []
    linvs = []
    for p, (f0, nf) in enumerate(_QS):
        splits = _make_plane_split(_Q_BLK0[p], _Q_NBLK[p])(W_emb.T, W_lin.T)
        et, lv = _make_sc_gather(nf)(xts[p], *splits)
        embts.append(et)
        linvs.append(lv)

    n_blk = _B // _RB
    e_specs = [pl.BlockSpec((nf * _D, _RB), lambda i: (0, i))
               for _, nf in _QS]
    l_specs = [pl.BlockSpec((nf, _RB), lambda i: (0, i)) for _, nf in _QS]
    a, d = pl.pallas_call(
        _tc_head,
        grid=(n_blk,),
        in_specs=e_specs + l_specs + [
            pl.BlockSpec((128, _NF * _D), lambda i: (0, 0)),
            pl.BlockSpec((128, 1), lambda i: (0, 0)),
            pl.BlockSpec((64, 128), lambda i: (0, 0)),
            pl.BlockSpec((64, 1), lambda i: (0, 0)),
            pl.BlockSpec((1, 64), lambda i: (0, 0)),
        ],
        out_specs=[
            pl.BlockSpec((1, _RB), lambda i: (0, i)),
            pl.BlockSpec((1, _RB), lambda i: (0, i)),
        ],
        out_shape=[
            jax.ShapeDtypeStruct((1, _B), jnp.float32),
            jax.ShapeDtypeStruct((1, _B), jnp.float32),
        ],
    )(*embts, *linvs, w0f, b0f, w1f, b1f, Wout)

    ue = jnp.exp(-a).reshape(_B, 1)
    dt = jnp.exp(-(d + bias_all))

    out = pl.pallas_call(
        _tc_out,
        grid=(n_blk,),
        in_specs=[
            pl.BlockSpec((_RB, 1), lambda i: (i, 0)),
            pl.BlockSpec((1, _B), lambda i: (0, 0)),
        ],
        out_specs=pl.BlockSpec((_RB, _B), lambda i: (i, 0)),
        out_shape=jax.ShapeDtypeStruct((_B, _B), jnp.float32),
    )(ue, dt)
    return out


# speedup vs baseline: 13.7806x; 1.0017x over previous
import functools

import jax
import jax.numpy as jnp
import numpy as np
from jax import lax
from jax.experimental import pallas as pl
from jax.experimental.pallas import tpu as pltpu
from jax.experimental.pallas import tpu_sc as plsc

_FIELD_DIMS = [100000] * 26
_OFFS = np.array((0, *np.cumsum(_FIELD_DIMS)[:-1]), dtype=np.int32)
_B = 4096
_NF = 26
_D = 16
_EPS = 1e-5

_NC = 2
_NS = 16
_NW = _NC * _NS
_RPW = _B // _NW


def _sc_gather_body(nf, idx_hbm, *refs):
    planes = refs[:_D]
    lin_hbm = refs[_D]
    emb_out, lin_out = refs[_D + 1], refs[_D + 2]
    idx_v, ebuf, lbuf, sem_e, sem_l = refs[_D + 3:]

    wid = lax.axis_index("s") * _NC + lax.axis_index("c")
    cbase = pl.multiple_of(wid * _RPW, 8)
    pltpu.sync_copy(idx_hbm.at[:, pl.ds(cbase, _RPW)], idx_v)

    def fire(f, carry):
        idx_row = idx_v.at[f]
        for d in range(_D):
            pltpu.make_async_copy(
                planes[d].at[idx_row], ebuf.at[f * _D + d], sem_e
            ).start()
        pltpu.make_async_copy(
            lin_hbm.at[idx_row], lbuf.at[f], sem_l
        ).start()
        return carry

    lax.fori_loop(0, nf, fire, 0)

    pltpu.make_async_copy(
        emb_out.at[:, pl.ds(0, _RPW)], ebuf, sem_e
    ).wait()
    pltpu.make_async_copy(
        lin_out.at[:, pl.ds(0, _RPW)], lbuf, sem_l
    ).wait()

    pltpu.sync_copy(ebuf, emb_out.at[:, pl.ds(cbase, _RPW)])
    pltpu.sync_copy(lbuf, lin_out.at[:, pl.ds(cbase, _RPW)])


@functools.lru_cache(maxsize=4)
def _make_sc_gather(nf):
    mesh = plsc.VectorSubcoreMesh(
        core_axis_name="c", subcore_axis_name="s",
        num_cores=_NC, num_subcores=_NS,
    )
    return pl.kernel(
        functools.partial(_sc_gather_body, nf),
        out_type=(
            jax.ShapeDtypeStruct((nf * _D, _B), jnp.float32),
            jax.ShapeDtypeStruct((nf, _B), jnp.float32),
        ),
        mesh=mesh,
        compiler_params=pltpu.CompilerParams(needs_layout_passes=False),
        scratch_types=[
            pltpu.VMEM((nf, _RPW), jnp.int32),
            pltpu.VMEM((nf * _D, _RPW), jnp.float32),
            pltpu.VMEM((nf, _RPW), jnp.float32),
            pltpu.SemaphoreType.DMA,
            pltpu.SemaphoreType.DMA,
        ],
    )


def _tc_split(emb_ref, lin_ref, *out_refs):
    e = emb_ref[...]
    for d in range(_D):
        out_refs[d][...] = e[d, :]
    out_refs[_D][...] = lin_ref[0, :]


_CB = 16384
_QS = [(0, 7), (7, 6), (13, 7), (20, 6)]
_Q_START = [f0 * _FIELD_DIMS[0] for f0, _ in _QS]
_Q_END = [(f0 + nf) * _FIELD_DIMS[0] for f0, nf in _QS]
_Q_BLK0 = [s // _CB for s in _Q_START]
_Q_OFF = [b * _CB for b in _Q_BLK0]
_Q_NBLK = [(e - o + _CB - 1) // _CB for e, o in zip(_Q_END, _Q_OFF)]


@functools.lru_cache(maxsize=4)
def _make_plane_split(blk0, n_blk):
    vh = n_blk * _CB
    return pl.pallas_call(
        _tc_split,
        grid=(n_blk,),
        in_specs=[
            pl.BlockSpec((_D, _CB), lambda i: (0, i + blk0)),
            pl.BlockSpec((1, _CB), lambda i: (0, i + blk0)),
        ],
        out_specs=[pl.BlockSpec((_CB,), lambda i: (i,))] * (_D + 1),
        out_shape=[jax.ShapeDtypeStruct((vh,), jnp.float32)] * (_D + 1),
    )


_RB = 512


def _tc_head(e1_ref, e2_ref, e3_ref, e4_ref, l1_ref, l2_ref, l3_ref, l4_ref,
             w0_ref, b0_ref, w1_ref, b1_ref, wo_ref, a_ref, d_ref):
    e_refs = (e1_ref, e2_ref, e3_ref, e4_ref)
    l_refs = (l1_ref, l2_ref, l3_ref, l4_ref)
    w0 = w0_ref[...]
    s = None
    q = None
    lin = None
    h = None
    col = 0
    for p, (_, nf) in enumerate(_QS):
        e = e_refs[p][...]
        for f in range(nf):
            c = e[f * _D:(f + 1) * _D, :]
            s = c if s is None else s + c
            q = c * c if q is None else q + c * c
        lp = jnp.sum(l_refs[p][...], axis=0, keepdims=True)
        lin = lp if lin is None else lin + lp
        hp = jnp.dot(w0[:, col:col + nf * _D], e,
                     preferred_element_type=jnp.float32)
        h = hp if h is None else h + hp
        col += nf * _D
    inter = 0.5 * jnp.sum(s * s - q, axis=0, keepdims=True)
    a_ref[...] = lin + inter
    h = jnp.maximum(h + b0_ref[...], 0.0)
    h = jnp.dot(w1_ref[...], h, preferred_element_type=jnp.float32)
    h = jnp.maximum(h + b1_ref[...], 0.0)
    d_ref[...] = jnp.dot(wo_ref[...], h, preferred_element_type=jnp.float32)


def _tc_out(a_ref, dt_ref, out_ref):
    out_ref[...] = 1.0 / (1.0 + a_ref[...] * dt_ref[...])


def kernel(x, W_emb, W_lin, lin_bias, W0, b0, g0, be0, rm0, rv0,
           W1, b1, g1, be1, rm1, rv1, Wout, bout):
    offs = jnp.asarray(_OFFS, dtype=x.dtype)
    xts = [
        x.T[f0:f0 + nf] + (offs[f0:f0 + nf, None] - _Q_OFF[p])
        for p, (f0, nf) in enumerate(_QS)
    ]

    inv0 = g0 / jnp.sqrt(rv0 + _EPS)
    w0f = W0 * inv0[:, None]
    b0f = ((b0 - rm0) * inv0 + be0).reshape(-1, 1)
    inv1 = g1 / jnp.sqrt(rv1 + _EPS)
    w1f = W1 * inv1[:, None]
    b1f = ((b1 - rm1) * inv1 + be1).reshape(-1, 1)
    bias_all = (bout + lin_bias)[0]

    embts = []
    linvs = []
    for p, (f0, nf) in enumerate(_QS):
        splits = _make_plane_split(_Q_BLK0[p], _Q_NBLK[p])(W_emb.T, W_lin.T)
        et, lv = _make_sc_gather(nf)(xts[p], *splits)
        embts.append(et)
        linvs.append(lv)

    n_blk = _B // _RB
    e_specs = [pl.BlockSpec((nf * _D, _RB), lambda i: (0, i))
               for _, nf in _QS]
    l_specs = [pl.BlockSpec((nf, _RB), lambda i: (0, i)) for _, nf in _QS]
    a, d = pl.pallas_call(
        _tc_head,
        grid=(n_blk,),
        in_specs=e_specs + l_specs + [
            pl.BlockSpec((128, _NF * _D), lambda i: (0, 0)),
            pl.BlockSpec((128, 1), lambda i: (0, 0)),
            pl.BlockSpec((64, 128), lambda i: (0, 0)),
            pl.BlockSpec((64, 1), lambda i: (0, 0)),
            pl.BlockSpec((1, 64), lambda i: (0, 0)),
        ],
        out_specs=[
            pl.BlockSpec((1, _RB), lambda i: (0, i)),
            pl.BlockSpec((1, _RB), lambda i: (0, i)),
        ],
        out_shape=[
            jax.ShapeDtypeStruct((1, _B), jnp.float32),
            jax.ShapeDtypeStruct((1, _B), jnp.float32),
        ],
    )(*embts, *linvs, w0f, b0f, w1f, b1f, Wout)

    ue = jnp.exp(-a).reshape(_B, 1)
    dt = jnp.exp(-(d + bias_all))

    out = pl.pallas_call(
        _tc_out,
        grid=(n_blk,),
        in_specs=[
            pl.BlockSpec((_RB, 1), lambda i: (i, 0)),
            pl.BlockSpec((1, _B), lambda i: (0, 0)),
        ],
        out_specs=pl.BlockSpec((_RB, _B), lambda i: (i, 0)),
        out_shape=jax.ShapeDtypeStruct((_B, _B), jnp.float32),
    )(ue, dt)
    return out
